# speedup vs baseline: 3.3255x; 3.3255x over previous
import functools

import jax
import jax.numpy as jnp
from jax import lax
from jax.experimental import pallas as pl
from jax.experimental.pallas import tpu as pltpu
from jax.experimental.pallas import tpu_sc as plsc

_D = 128
_CH = 128
_NBUF = 2


@functools.lru_cache(maxsize=None)
def _build(B):
    info = plsc.get_sparse_core_info()
    NC, NS = info.num_cores, info.num_subcores
    NW = NC * NS
    assert B % (NW * _CH) == 0
    BPW = B // NW
    NCH = BPW // _CH
    assert NCH % _NBUF == 0
    NOUT = NCH // _NBUF

    mesh = plsc.VectorSubcoreMesh(core_axis_name="c", subcore_axis_name="s")

    @functools.partial(
        pl.kernel,
        mesh=mesh,
        out_type=jax.ShapeDtypeStruct((B, _D), jnp.float32),
        scratch_types=[
            pltpu.VMEM((NCH, _CH), jnp.int32),
            pltpu.VMEM((_NBUF, _CH, _D), jnp.float32),
            pltpu.SemaphoreType.DMA((_NBUF,)),
            pltpu.SemaphoreType.DMA((_NBUF,)),
        ],
    )
    def emb(table_hbm, ids_hbm, out_hbm, idx_v, bufs, gsem, osem):
        wid = lax.axis_index("s") * NC + lax.axis_index("c")
        base = wid * BPW
        pltpu.sync_copy(ids_hbm.at[wid], idx_v)
        for b in range(_NBUF):
            pltpu.async_copy(table_hbm.at[idx_v.at[b]], bufs.at[b], gsem.at[b])

        def outer(g, carry):
            for b in range(_NBUF):
                j = g * _NBUF + b
                row = base + j * _CH
                pltpu.make_async_copy(
                    table_hbm.at[idx_v.at[j]], bufs.at[b], gsem.at[b]
                ).wait()
                pltpu.async_copy(
                    bufs.at[b], out_hbm.at[pl.ds(row, _CH)], osem.at[b]
                )

                @pl.when(g < NOUT - 1)
                def _():
                    pltpu.make_async_copy(
                        bufs.at[b], out_hbm.at[pl.ds(row, _CH)], osem.at[b]
                    ).wait()
                    pltpu.async_copy(
                        table_hbm.at[idx_v.at[j + _NBUF]], bufs.at[b], gsem.at[b]
                    )
            return carry

        lax.fori_loop(0, NOUT, outer, 0)
        for b in range(_NBUF):
            j = (NOUT - 1) * _NBUF + b
            row = base + j * _CH
            pltpu.make_async_copy(
                bufs.at[b], out_hbm.at[pl.ds(row, _CH)], osem.at[b]
            ).wait()

    return emb, NW, NCH


def kernel(token_ids, embedding):
    B = token_ids.size
    emb, NW, NCH = _build(B)
    ids = token_ids.reshape(NW, NCH, _CH).astype(jnp.int32)
    out = emb(embedding, ids)
    return out.reshape(token_ids.shape + (_D,))


# speedup vs baseline: 3.3467x; 1.0064x over previous
import functools

import jax
import jax.numpy as jnp
from jax import lax
from jax.experimental import pallas as pl
from jax.experimental.pallas import tpu as pltpu
from jax.experimental.pallas import tpu_sc as plsc

_D = 128
_CH = 128
_NBUF = 5


@functools.lru_cache(maxsize=None)
def _build(B):
    info = plsc.get_sparse_core_info()
    NC, NS = info.num_cores, info.num_subcores
    NW = NC * NS
    assert B % (NW * _CH) == 0
    BPW = B // NW
    NCH = BPW // _CH
    assert NCH % _NBUF == 0
    NOUT = NCH // _NBUF

    mesh = plsc.VectorSubcoreMesh(core_axis_name="c", subcore_axis_name="s")

    @functools.partial(
        pl.kernel,
        mesh=mesh,
        out_type=jax.ShapeDtypeStruct((B, _D), jnp.float32),
        scratch_types=[
            pltpu.VMEM((NCH, _CH), jnp.int32),
            pltpu.VMEM((_NBUF, _CH, _D), jnp.float32),
            pltpu.SemaphoreType.DMA((_NBUF,)),
            pltpu.SemaphoreType.DMA((_NBUF,)),
        ],
    )
    def emb(table_hbm, ids_hbm, out_hbm, idx_v, bufs, gsem, osem):
        wid = lax.axis_index("s") * NC + lax.axis_index("c")
        base = wid * BPW
        pltpu.sync_copy(ids_hbm.at[wid], idx_v)
        for b in range(_NBUF):
            pltpu.async_copy(table_hbm.at[idx_v.at[b]], bufs.at[b], gsem.at[b])

        def outer(g, carry):
            for b in range(_NBUF):
                j = g * _NBUF + b
                row = base + j * _CH
                pltpu.make_async_copy(
                    table_hbm.at[idx_v.at[j]], bufs.at[b], gsem.at[b]
                ).wait()
                pltpu.async_copy(
                    bufs.at[b], out_hbm.at[pl.ds(row, _CH)], osem.at[b]
                )

                @pl.when(g < NOUT - 1)
                def _():
                    pltpu.make_async_copy(
                        bufs.at[b], out_hbm.at[pl.ds(row, _CH)], osem.at[b]
                    ).wait()
                    pltpu.async_copy(
                        table_hbm.at[idx_v.at[j + _NBUF]], bufs.at[b], gsem.at[b]
                    )
            return carry

        lax.fori_loop(0, NOUT, outer, 0)
        for b in range(_NBUF):
            j = (NOUT - 1) * _NBUF + b
            row = base + j * _CH
            pltpu.make_async_copy(
                bufs.at[b], out_hbm.at[pl.ds(row, _CH)], osem.at[b]
            ).wait()

    return emb, NW, NCH


def kernel(token_ids, embedding):
    B = token_ids.size
    emb, NW, NCH = _build(B)
    ids = token_ids.reshape(NW, NCH, _CH).astype(jnp.int32)
    out = emb(embedding, ids)
    return out.reshape(token_ids.shape + (_D,))


# speedup vs baseline: 5.9553x; 1.7794x over previous
import functools

import jax
import jax.numpy as jnp
from jax import lax
from jax.experimental import pallas as pl
from jax.experimental.pallas import tpu as pltpu
from jax.experimental.pallas import tpu_sc as plsc

_D = 128
_SEQ_PER_CH = 2
_NBUF = 4


@functools.lru_cache(maxsize=None)
def _build(n_seq, seq_len):
    info = plsc.get_sparse_core_info()
    NC, NS = info.num_cores, info.num_subcores
    NW = NC * NS
    CH = _SEQ_PER_CH * seq_len
    assert n_seq % (NW * _SEQ_PER_CH) == 0
    SPW = n_seq // NW
    NCH = SPW // _SEQ_PER_CH
    assert NCH % _NBUF == 0
    NOUT = NCH // _NBUF

    mesh = plsc.VectorSubcoreMesh(core_axis_name="c", subcore_axis_name="s")

    @functools.partial(
        pl.kernel,
        mesh=mesh,
        out_type=jax.ShapeDtypeStruct((n_seq, seq_len, _D), jnp.float32),
        scratch_types=[
            pltpu.VMEM((NCH, CH), jnp.int32),
            pltpu.VMEM((_NBUF, CH, _D), jnp.float32),
            pltpu.SemaphoreType.DMA((_NBUF,)),
            pltpu.SemaphoreType.DMA((_NBUF,)),
        ],
    )
    def emb(table_hbm, ids_hbm, out_hbm, idx_v, bufs, gsem, osem):
        wid = lax.axis_index("s") * NC + lax.axis_index("c")
        seq_base = wid * SPW
        pltpu.sync_copy(ids_hbm.at[wid], idx_v)
        for b in range(_NBUF):
            pltpu.async_copy(table_hbm.at[idx_v.at[b]], bufs.at[b], gsem.at[b])

        def chunk_writes(j, b):
            seq = seq_base + j * _SEQ_PER_CH
            for q in range(_SEQ_PER_CH):
                yield (bufs.at[b].at[pl.ds(q * seq_len, seq_len)],
                       out_hbm.at[seq + q])

        def outer(g, carry):
            for b in range(_NBUF):
                j = g * _NBUF + b
                pltpu.make_async_copy(
                    table_hbm.at[idx_v.at[j]], bufs.at[b], gsem.at[b]
                ).wait()
                for src, dst in chunk_writes(j, b):
                    pltpu.async_copy(src, dst, osem.at[b])

                @pl.when(g < NOUT - 1)
                def _():
                    for src, dst in chunk_writes(j, b):
                        pltpu.make_async_copy(src, dst, osem.at[b]).wait()
                    pltpu.async_copy(
                        table_hbm.at[idx_v.at[j + _NBUF]], bufs.at[b], gsem.at[b]
                    )
            return carry

        lax.fori_loop(0, NOUT, outer, 0)
        for b in range(_NBUF):
            j = (NOUT - 1) * _NBUF + b
            for src, dst in chunk_writes(j, b):
                pltpu.make_async_copy(src, dst, osem.at[b]).wait()

    return emb, NW, NCH, CH


def kernel(token_ids, embedding):
    n_seq, seq_len = token_ids.shape
    emb, NW, NCH, CH = _build(n_seq, seq_len)
    ids = token_ids.reshape(NW, NCH, CH).astype(jnp.int32)
    return emb(embedding, ids)
